# baseline (speedup 1.0000x reference)
import jax
import jax.numpy as jnp
from jax import lax
from jax.experimental import pallas as pl
from jax.experimental.pallas import tpu as pltpu
from jax.experimental.pallas import tpu_sc as plsc

NUM_WORKERS = 32
ROWS = 4096
HIST = 200
D = 64
C = ROWS // NUM_WORKERS
TP = C + 1
NBUF = 4
LEAD = 3
NT = 2

assert HIST % NBUF == 0 and LEAD < NBUF


def _body(ids_hbm, table_hbm, out_hbm, idx_v, bufs, tbufs, sem_g, sem_w):
    wid = lax.axis_index("s") * 2 + lax.axis_index("c")
    base = wid * C

    pltpu.sync_copy(ids_hbm.at[:, pl.ds(base, C)], idx_v)

    def gather(j, b):
        pltpu.async_copy(table_hbm.at[idx_v.at[j]], bufs.at[b], sem_g)

    def writeback(j, t):
        pltpu.async_copy(tbufs.at[t, :, 0:C], out_hbm.at[j, :, pl.ds(base, C)],
                         sem_w)

    def wait_g():
        pltpu.make_async_copy(table_hbm.at[idx_v.at[0]], bufs.at[0],
                              sem_g).wait()

    def wait_w():
        pltpu.make_async_copy(tbufs.at[0, :, 0:C],
                              out_hbm.at[0, :, pl.ds(base, C)], sem_w).wait()

    iota = lax.iota(jnp.int32, 16)
    col_idx = [iota + 16 * k for k in range(D // 16)]

    def transpose(b, t):
        def tr_row(r, carry):
            row_idx = jnp.full((16,), r, jnp.int32)
            for k in range(D // 16):
                x = bufs[b, r, pl.ds(16 * k, 16)]
                plsc.store_scatter(tbufs.at[t], [col_idx[k], row_idx], x)
            return carry

        lax.fori_loop(0, C, tr_row, 0, unroll=2)

    for i in range(LEAD):
        gather(i, i)
    for b in range(NBUF):
        gather(b + LEAD, (b + LEAD) % NBUF)
        wait_g()
        if b >= NT:
            wait_w()
        transpose(b, b % NT)
        writeback(b, b % NT)

    def step(g, carry):
        j0 = g * NBUF
        for b in range(NBUF):
            gather(j0 + b + LEAD, (b + LEAD) % NBUF)
            wait_g()
            wait_w()
            transpose((b) % NBUF, b % NT)
            writeback(j0 + b, b % NT)
        return carry

    lax.fori_loop(1, HIST // NBUF - 1, step, 0)

    j0 = HIST - NBUF
    for b in range(NBUF):
        if b < NBUF - LEAD:
            gather(j0 + b + LEAD, (b + LEAD) % NBUF)
        wait_g()
        wait_w()
        transpose(b, b % NT)
        writeback(j0 + b, b % NT)
    for _ in range(NT):
        wait_w()


def kernel(token_ids, weights):
    run = pl.kernel(
        _body,
        out_type=jax.ShapeDtypeStruct((HIST, D, ROWS), jnp.float32),
        mesh=plsc.VectorSubcoreMesh(core_axis_name="c", subcore_axis_name="s"),
        scratch_types=[
            pltpu.VMEM((HIST, C), jnp.int32),
            pltpu.VMEM((NBUF, C, D), jnp.float32),
            pltpu.VMEM((NT, D, TP), jnp.float32),
            pltpu.SemaphoreType.DMA,
            pltpu.SemaphoreType.DMA,
        ],
        compiler_params=pltpu.CompilerParams(use_tc_tiling_on_sc=False,
                                             needs_layout_passes=False),
    )
    out_p = run(token_ids.T.astype(jnp.int32), weights)
    return out_p.transpose(2, 0, 1)
